# Initial kernel scaffold; baseline (speedup 1.0000x reference)
import jax
import jax.numpy as jnp
from jax.experimental import pallas as pl


def kernel(user_positive_items_pairs, negative_samples, textual_feature_pos, visual_feature_pos, textual_feature_neg, visual_feature_neg, user_table, item_table, edge_index, edge_weight, t1_W, t1_b, t2_W, t2_b, v1_W, v1_b, v2_W, v2_b):
    raise NotImplementedError("write your pallas kernel here")


# speedup vs baseline: 3.6694x; 3.6694x over previous
import functools

import jax
import jax.numpy as jnp
from jax import lax
from jax.experimental import pallas as pl
from jax.experimental.pallas import tpu as pltpu
from jax.experimental.pallas import tpu_sc as plsc

N_USERS = 50000
N_ITEMS = 50000
N_NODES = N_USERS + N_ITEMS
N_EDGES = 3200000
D = 20
NUM_NEG = 4
N_LAYERS = 3
HID = 512

NC = 2
NS = 16
NW = NC * NS
L = 16

DP = 32
K = 768
KB = K // 128
NCHUNK = 261
EW = NCHUNK * K
E_PAD = NS * EW
NACC = 100352
HALF = NACC // 2
NACCH = 50304
DUMMY = HALF
SLAB = NACCH // NS

B_GATHER = 24576
GPW = B_GATHER // NW
GB = GPW // 128


def _spmm_body(table, srcb, dstb, wflat, zeros, out,
               idx_v, dsti_v, w_v, rows_v, acc, sem_g, sem_s):
    cid = lax.axis_index("c")
    sid = lax.axis_index("s")
    lanes = lax.iota(jnp.int32, 16)
    lo = cid * HALF

    pltpu.sync_copy(zeros.at[pl.ds(sid * SLAB, SLAB)],
                    acc.at[pl.ds(sid * SLAB, SLAB)])
    plsc.subcore_barrier()

    def chunk_body(ck, carry):
        row0 = sid * (NCHUNK * KB) + ck * KB
        e0 = sid * EW + ck * K
        pltpu.sync_copy(srcb.at[pl.ds(row0, KB)], idx_v)
        pltpu.sync_copy(dstb.at[pl.ds(row0, KB)], dsti_v)
        pltpu.sync_copy(wflat.at[pl.ds(e0, K)], w_v)
        descs = [
            pltpu.async_copy(table.at[idx_v.at[j]],
                             rows_v.at[pl.ds(j * 128, 128)], sem_g)
            for j in range(KB)
        ]
        for dsc in descs:
            dsc.wait()

        def scale_body(g, c2):
            eb = g * 16
            rj = jnp.full((16,), g // 8, jnp.int32)
            cj = (g % 8) * 16 + lanes
            dv = plsc.load_gather(dsti_v, [rj, cj])
            loc = dv - lo
            ok = (loc >= 0) & (loc < HALF)
            plsc.store_scatter(dsti_v, [rj, cj], jnp.where(ok, loc, DUMMY))
            ei = eb + lanes
            wv = w_v[pl.ds(eb, 16)]
            for c in range(16):
                cd = jnp.full((16,), c, jnp.int32)
                v = plsc.load_gather(rows_v, [ei, cd])
                plsc.store_scatter(rows_v, [ei, cd], v * wv)
            for t in range(4):
                rt = eb + t * 4 + lanes // 4
                ct = 16 + (lanes % 4)
                wt = plsc.load_gather(w_v, [rt])
                v = plsc.load_gather(rows_v, [rt, ct])
                plsc.store_scatter(rows_v, [rt, ct], v * wt)
            return c2

        lax.fori_loop(0, K // 16, scale_body, 0)

        descs2 = [
            pltpu.async_copy(rows_v.at[pl.ds(j * 128, 128)],
                             acc.at[dsti_v.at[j]], sem_s, add=True)
            for j in range(KB)
        ]
        for dsc in descs2:
            dsc.wait()
        return carry

    lax.fori_loop(0, NCHUNK, chunk_body, 0)
    plsc.subcore_barrier()
    pltpu.sync_copy(acc.at[pl.ds(sid * SLAB, SLAB)],
                    out.at[cid, pl.ds(sid * SLAB, SLAB)])


_spmm = pl.kernel(
    _spmm_body,
    out_type=jax.ShapeDtypeStruct((NC, NACCH, DP), jnp.float32),
    mesh=plsc.VectorSubcoreMesh(core_axis_name="c", subcore_axis_name="s"),
    compiler_params=pltpu.CompilerParams(needs_layout_passes=False,
                                         use_tc_tiling_on_sc=False),
    scratch_types=[
        pltpu.VMEM((KB, 128), jnp.int32),
        pltpu.VMEM((KB, 128), jnp.int32),
        pltpu.VMEM((K,), jnp.float32),
        pltpu.VMEM((K, DP), jnp.float32),
        pltpu.VMEM_SHARED((NACCH, DP), jnp.float32),
        pltpu.SemaphoreType.DMA,
        pltpu.SemaphoreType.DMA,
    ],
)


def _gather_body(light, idxb, out, idx_v, rows_v, sem):
    cid = lax.axis_index("c")
    sid = lax.axis_index("s")
    wid = cid * NS + sid
    pltpu.sync_copy(idxb.at[pl.ds(wid * GB, GB)], idx_v)
    descs = [
        pltpu.async_copy(light.at[idx_v.at[j]],
                         rows_v.at[pl.ds(j * 128, 128)], sem)
        for j in range(GB)
    ]
    for dsc in descs:
        dsc.wait()
    pltpu.sync_copy(rows_v, out.at[pl.ds(wid * GPW, GPW)])


_gather = pl.kernel(
    _gather_body,
    out_type=jax.ShapeDtypeStruct((B_GATHER, DP), jnp.float32),
    mesh=plsc.VectorSubcoreMesh(core_axis_name="c", subcore_axis_name="s"),
    compiler_params=pltpu.CompilerParams(needs_layout_passes=False, use_tc_tiling_on_sc=False),
    scratch_types=[
        pltpu.VMEM((GB, 128), jnp.int32),
        pltpu.VMEM((GPW, DP), jnp.float32),
        pltpu.SemaphoreType.DMA,
    ],
)


def _add2_body(p_ref, r_ref, t_ref, ro_ref):
    s = p_ref[0]
    t_ref[...] = s
    ro_ref[...] = r_ref[...] + s


def _addfin_body(p_ref, r_ref, o_ref):
    o_ref[...] = (r_ref[...] + p_ref[0]) * 0.25


_ADD_BR = 784
_ADD_GRID = NACC // _ADD_BR
_HB = HALF // _ADD_BR


def _add2(partials, run):
    return pl.pallas_call(
        _add2_body,
        grid=(_ADD_GRID,),
        in_specs=[
            pl.BlockSpec((1, _ADD_BR, DP), lambda i: (i // _HB, i % _HB, 0)),
            pl.BlockSpec((_ADD_BR, DP), lambda i: (i, 0)),
        ],
        out_specs=[
            pl.BlockSpec((_ADD_BR, DP), lambda i: (i, 0)),
            pl.BlockSpec((_ADD_BR, DP), lambda i: (i, 0)),
        ],
        out_shape=[
            jax.ShapeDtypeStruct((NACC, DP), jnp.float32),
            jax.ShapeDtypeStruct((NACC, DP), jnp.float32),
        ],
    )(partials, run)


def _addfin(partials, run):
    return pl.pallas_call(
        _addfin_body,
        grid=(_ADD_GRID,),
        in_specs=[
            pl.BlockSpec((1, _ADD_BR, DP), lambda i: (i // _HB, i % _HB, 0)),
            pl.BlockSpec((_ADD_BR, DP), lambda i: (i, 0)),
        ],
        out_specs=pl.BlockSpec((_ADD_BR, DP), lambda i: (i, 0)),
        out_shape=jax.ShapeDtypeStruct((NACC, DP), jnp.float32),
    )(partials, run)


def _mlp_body(x_ref, w1_ref, b1_ref, w2_ref, b2_ref, o_ref):
    x = x_ref[...]
    n = jnp.sqrt(jnp.sum(x * x, axis=1, keepdims=True))
    f = x / jnp.maximum(n, 1e-12)
    h = jnp.dot(f, w1_ref[...], preferred_element_type=jnp.float32)
    h = h + b1_ref[...]
    h = jnp.where(h >= 0, h, 0.2 * h)
    n2 = jnp.sqrt(jnp.sum(h * h, axis=1, keepdims=True))
    g = h / jnp.maximum(n2, 1e-12)
    o_ref[...] = jnp.dot(g, w2_ref[...],
                         preferred_element_type=jnp.float32) + b2_ref[...]


def _mlp(x, w1, b1, w2, b2, bm=512):
    rows = x.shape[0]
    return pl.pallas_call(
        _mlp_body,
        grid=(rows // bm,),
        in_specs=[
            pl.BlockSpec((bm, 1024), lambda i: (i, 0)),
            pl.BlockSpec((1024, HID), lambda i: (0, 0)),
            pl.BlockSpec((1, HID), lambda i: (0, 0)),
            pl.BlockSpec((HID, D), lambda i: (0, 0)),
            pl.BlockSpec((1, D), lambda i: (0, 0)),
        ],
        out_specs=pl.BlockSpec((bm, D), lambda i: (i, 0)),
        out_shape=jax.ShapeDtypeStruct((rows, D), jnp.float32),
    )(x, w1, b1.reshape(1, -1), w2, b2.reshape(1, -1))


def kernel(user_positive_items_pairs, negative_samples, textual_feature_pos,
           visual_feature_pos, textual_feature_neg, visual_feature_neg,
           user_table, item_table, edge_index, edge_weight,
           t1_W, t1_b, t2_W, t2_b, v1_W, v1_b, v2_W, v2_b):
    emb0 = jnp.concatenate(
        [user_table, item_table,
         jnp.zeros((NACC - N_NODES, D), jnp.float32)], axis=0)
    emb0 = jnp.pad(emb0, ((0, 0), (0, DP - D)))

    pad = E_PAD - N_EDGES
    srcb = jnp.concatenate(
        [edge_index[0], jnp.zeros((pad,), jnp.int32)]).reshape(-1, 128)
    dstb = jnp.concatenate(
        [edge_index[1], jnp.full((pad,), N_NODES, jnp.int32)]).reshape(-1, 128)
    wflat = jnp.concatenate([edge_weight, jnp.zeros((pad,), jnp.float32)])
    zeros = jnp.zeros((NACCH, DP), jnp.float32)

    table = emb0
    run = emb0
    light = None
    for layer in range(N_LAYERS):
        partials = _spmm(table, srcb, dstb, wflat, zeros)
        if layer < N_LAYERS - 1:
            table, run = _add2(partials, run)
        else:
            light = _addfin(partials, run)

    u_idx = user_positive_items_pairs[:, 0]
    p_idx = user_positive_items_pairs[:, 1] + N_USERS
    n_idx = negative_samples.reshape(-1) + N_USERS
    idx_all = jnp.concatenate([u_idx, p_idx, n_idx]).reshape(-1, 128)
    g24 = _gather(light, idx_all)

    g24 = g24[:, :D]
    users = g24[:4096]
    pos_items = g24[4096:8192]
    neg_items = g24[8192:]

    pos_t = _mlp(textual_feature_pos, t1_W, t1_b, t2_W, t2_b)
    neg_t = _mlp(textual_feature_neg.reshape(-1, 1024), t1_W, t1_b, t2_W, t2_b)
    pos_v = _mlp(visual_feature_pos, v1_W, v1_b, v2_W, v2_b)
    neg_v = _mlp(visual_feature_neg.reshape(-1, 1024), v1_W, v1_b, v2_W, v2_b)

    items = jnp.concatenate([pos_items, neg_items], axis=0)
    textual_f = jnp.concatenate([pos_t, neg_t], axis=0)
    visual_f = jnp.concatenate([pos_v, neg_v], axis=0)
    user_a_ = jnp.repeat(users[:, None, :], NUM_NEG, axis=1).reshape(-1, D)
    users_all = jnp.concatenate([users, user_a_], axis=0)
    return jnp.stack([users_all, items, textual_f, visual_f], axis=0)


# speedup vs baseline: 3.6852x; 1.0043x over previous
import functools

import jax
import jax.numpy as jnp
from jax import lax
from jax.experimental import pallas as pl
from jax.experimental.pallas import tpu as pltpu
from jax.experimental.pallas import tpu_sc as plsc

N_USERS = 50000
N_ITEMS = 50000
N_NODES = N_USERS + N_ITEMS
N_EDGES = 3200000
D = 20
NUM_NEG = 4
N_LAYERS = 3
HID = 512

NC = 2
NS = 16
NW = NC * NS
L = 16

DP = 32
K = 768
KB = K // 128
NCHUNK = 261
EW = NCHUNK * K
E_PAD = NS * EW
NACC = 100352
HALF = NACC // 2
NACCH = 50304
DUMMY = HALF
SLAB = NACCH // NS

B_GATHER = 24576
GPW = B_GATHER // NW
GB = GPW // 128


def _spmm_body(table, srcb, dstb, wflat, zeros, out,
               idx_v, dsti_v, w_v, rows_v, acc, sem_g, sem_s):
    cid = lax.axis_index("c")
    sid = lax.axis_index("s")
    lanes = lax.iota(jnp.int32, 16)
    lo = cid * HALF

    pltpu.sync_copy(zeros.at[pl.ds(sid * SLAB, SLAB)],
                    acc.at[pl.ds(sid * SLAB, SLAB)])
    plsc.subcore_barrier()

    def chunk_body(ck, carry):
        row0 = sid * (NCHUNK * KB) + ck * KB
        e0 = sid * EW + ck * K
        pltpu.sync_copy(srcb.at[pl.ds(row0, KB)], idx_v)
        pltpu.sync_copy(dstb.at[pl.ds(row0, KB)], dsti_v)
        pltpu.sync_copy(wflat.at[pl.ds(e0, K)], w_v)
        descs = [
            pltpu.async_copy(table.at[idx_v.at[j]],
                             rows_v.at[pl.ds(j * 128, 128)], sem_g)
            for j in range(KB)
        ]
        for dsc in descs:
            dsc.wait()

        def scale_body(g, c2):
            eb = g * 16
            rj = jnp.full((16,), g // 8, jnp.int32)
            cj = (g % 8) * 16 + lanes
            dv = plsc.load_gather(dsti_v, [rj, cj])
            loc = dv - lo
            ok = (loc >= 0) & (loc < HALF)
            spread = DUMMY + (dv & 127)
            plsc.store_scatter(dsti_v, [rj, cj], jnp.where(ok, loc, spread))
            ei = eb + lanes
            wv = w_v[pl.ds(eb, 16)]
            for c in range(16):
                cd = jnp.full((16,), c, jnp.int32)
                v = plsc.load_gather(rows_v, [ei, cd])
                plsc.store_scatter(rows_v, [ei, cd], v * wv)
            for t in range(4):
                rt = eb + t * 4 + lanes // 4
                ct = 16 + (lanes % 4)
                wt = plsc.load_gather(w_v, [rt])
                v = plsc.load_gather(rows_v, [rt, ct])
                plsc.store_scatter(rows_v, [rt, ct], v * wt)
            return c2

        lax.fori_loop(0, K // 16, scale_body, 0)

        descs2 = [
            pltpu.async_copy(rows_v.at[pl.ds(j * 128, 128)],
                             acc.at[dsti_v.at[j]], sem_s, add=True)
            for j in range(KB)
        ]
        for dsc in descs2:
            dsc.wait()
        return carry

    lax.fori_loop(0, NCHUNK, chunk_body, 0)
    plsc.subcore_barrier()
    pltpu.sync_copy(acc.at[pl.ds(sid * SLAB, SLAB)],
                    out.at[cid, pl.ds(sid * SLAB, SLAB)])


_spmm = pl.kernel(
    _spmm_body,
    out_type=jax.ShapeDtypeStruct((NC, NACCH, DP), jnp.float32),
    mesh=plsc.VectorSubcoreMesh(core_axis_name="c", subcore_axis_name="s"),
    compiler_params=pltpu.CompilerParams(needs_layout_passes=False,
                                         use_tc_tiling_on_sc=False),
    scratch_types=[
        pltpu.VMEM((KB, 128), jnp.int32),
        pltpu.VMEM((KB, 128), jnp.int32),
        pltpu.VMEM((K,), jnp.float32),
        pltpu.VMEM((K, DP), jnp.float32),
        pltpu.VMEM_SHARED((NACCH, DP), jnp.float32),
        pltpu.SemaphoreType.DMA,
        pltpu.SemaphoreType.DMA,
    ],
)


def _gather_body(light, idxb, out, idx_v, rows_v, sem):
    cid = lax.axis_index("c")
    sid = lax.axis_index("s")
    wid = cid * NS + sid
    pltpu.sync_copy(idxb.at[pl.ds(wid * GB, GB)], idx_v)
    descs = [
        pltpu.async_copy(light.at[idx_v.at[j]],
                         rows_v.at[pl.ds(j * 128, 128)], sem)
        for j in range(GB)
    ]
    for dsc in descs:
        dsc.wait()
    pltpu.sync_copy(rows_v, out.at[pl.ds(wid * GPW, GPW)])


_gather = pl.kernel(
    _gather_body,
    out_type=jax.ShapeDtypeStruct((B_GATHER, DP), jnp.float32),
    mesh=plsc.VectorSubcoreMesh(core_axis_name="c", subcore_axis_name="s"),
    compiler_params=pltpu.CompilerParams(needs_layout_passes=False, use_tc_tiling_on_sc=False),
    scratch_types=[
        pltpu.VMEM((GB, 128), jnp.int32),
        pltpu.VMEM((GPW, DP), jnp.float32),
        pltpu.SemaphoreType.DMA,
    ],
)


def _add2_body(p_ref, r_ref, t_ref, ro_ref):
    s = p_ref[0]
    t_ref[...] = s
    ro_ref[...] = r_ref[...] + s


def _addfin_body(p_ref, r_ref, o_ref):
    o_ref[...] = (r_ref[...] + p_ref[0]) * 0.25


_ADD_BR = 784
_ADD_GRID = NACC // _ADD_BR
_HB = HALF // _ADD_BR


def _add2(partials, run):
    return pl.pallas_call(
        _add2_body,
        grid=(_ADD_GRID,),
        in_specs=[
            pl.BlockSpec((1, _ADD_BR, DP), lambda i: (i // _HB, i % _HB, 0)),
            pl.BlockSpec((_ADD_BR, DP), lambda i: (i, 0)),
        ],
        out_specs=[
            pl.BlockSpec((_ADD_BR, DP), lambda i: (i, 0)),
            pl.BlockSpec((_ADD_BR, DP), lambda i: (i, 0)),
        ],
        out_shape=[
            jax.ShapeDtypeStruct((NACC, DP), jnp.float32),
            jax.ShapeDtypeStruct((NACC, DP), jnp.float32),
        ],
    )(partials, run)


def _addfin(partials, run):
    return pl.pallas_call(
        _addfin_body,
        grid=(_ADD_GRID,),
        in_specs=[
            pl.BlockSpec((1, _ADD_BR, DP), lambda i: (i // _HB, i % _HB, 0)),
            pl.BlockSpec((_ADD_BR, DP), lambda i: (i, 0)),
        ],
        out_specs=pl.BlockSpec((_ADD_BR, DP), lambda i: (i, 0)),
        out_shape=jax.ShapeDtypeStruct((NACC, DP), jnp.float32),
    )(partials, run)


def _mlp_body(x_ref, w1_ref, b1_ref, w2_ref, b2_ref, o_ref):
    x = x_ref[...]
    n = jnp.sqrt(jnp.sum(x * x, axis=1, keepdims=True))
    f = x / jnp.maximum(n, 1e-12)
    h = jnp.dot(f, w1_ref[...], preferred_element_type=jnp.float32)
    h = h + b1_ref[...]
    h = jnp.where(h >= 0, h, 0.2 * h)
    n2 = jnp.sqrt(jnp.sum(h * h, axis=1, keepdims=True))
    g = h / jnp.maximum(n2, 1e-12)
    o_ref[...] = jnp.dot(g, w2_ref[...],
                         preferred_element_type=jnp.float32) + b2_ref[...]


def _mlp(x, w1, b1, w2, b2, bm=512):
    rows = x.shape[0]
    return pl.pallas_call(
        _mlp_body,
        grid=(rows // bm,),
        in_specs=[
            pl.BlockSpec((bm, 1024), lambda i: (i, 0)),
            pl.BlockSpec((1024, HID), lambda i: (0, 0)),
            pl.BlockSpec((1, HID), lambda i: (0, 0)),
            pl.BlockSpec((HID, D), lambda i: (0, 0)),
            pl.BlockSpec((1, D), lambda i: (0, 0)),
        ],
        out_specs=pl.BlockSpec((bm, D), lambda i: (i, 0)),
        out_shape=jax.ShapeDtypeStruct((rows, D), jnp.float32),
    )(x, w1, b1.reshape(1, -1), w2, b2.reshape(1, -1))


def kernel(user_positive_items_pairs, negative_samples, textual_feature_pos,
           visual_feature_pos, textual_feature_neg, visual_feature_neg,
           user_table, item_table, edge_index, edge_weight,
           t1_W, t1_b, t2_W, t2_b, v1_W, v1_b, v2_W, v2_b):
    emb0 = jnp.concatenate(
        [user_table, item_table,
         jnp.zeros((NACC - N_NODES, D), jnp.float32)], axis=0)
    emb0 = jnp.pad(emb0, ((0, 0), (0, DP - D)))

    pad = E_PAD - N_EDGES
    srcb = jnp.concatenate(
        [edge_index[0], jnp.zeros((pad,), jnp.int32)]).reshape(-1, 128)
    dstb = jnp.concatenate(
        [edge_index[1], jnp.full((pad,), N_NODES, jnp.int32)]).reshape(-1, 128)
    wflat = jnp.concatenate([edge_weight, jnp.zeros((pad,), jnp.float32)])
    zeros = jnp.zeros((NACCH, DP), jnp.float32)

    table = emb0
    run = emb0
    light = None
    for layer in range(N_LAYERS):
        partials = _spmm(table, srcb, dstb, wflat, zeros)
        if layer < N_LAYERS - 1:
            table, run = _add2(partials, run)
        else:
            light = _addfin(partials, run)

    u_idx = user_positive_items_pairs[:, 0]
    p_idx = user_positive_items_pairs[:, 1] + N_USERS
    n_idx = negative_samples.reshape(-1) + N_USERS
    idx_all = jnp.concatenate([u_idx, p_idx, n_idx]).reshape(-1, 128)
    g24 = _gather(light, idx_all)

    g24 = g24[:, :D]
    users = g24[:4096]
    pos_items = g24[4096:8192]
    neg_items = g24[8192:]

    pos_t = _mlp(textual_feature_pos, t1_W, t1_b, t2_W, t2_b)
    neg_t = _mlp(textual_feature_neg.reshape(-1, 1024), t1_W, t1_b, t2_W, t2_b)
    pos_v = _mlp(visual_feature_pos, v1_W, v1_b, v2_W, v2_b)
    neg_v = _mlp(visual_feature_neg.reshape(-1, 1024), v1_W, v1_b, v2_W, v2_b)

    items = jnp.concatenate([pos_items, neg_items], axis=0)
    textual_f = jnp.concatenate([pos_t, neg_t], axis=0)
    visual_f = jnp.concatenate([pos_v, neg_v], axis=0)
    user_a_ = jnp.repeat(users[:, None, :], NUM_NEG, axis=1).reshape(-1, D)
    users_all = jnp.concatenate([users, user_a_], axis=0)
    return jnp.stack([users_all, items, textual_f, visual_f], axis=0)


# speedup vs baseline: 11.0213x; 2.9907x over previous
import functools

import jax
import jax.numpy as jnp
from jax import lax
from jax.experimental import pallas as pl
from jax.experimental.pallas import tpu as pltpu
from jax.experimental.pallas import tpu_sc as plsc

N_USERS = 50000
N_ITEMS = 50000
N_NODES = N_USERS + N_ITEMS
N_EDGES = 3200000
D = 20
NUM_NEG = 4
N_LAYERS = 3
HID = 512

NC = 2
NS = 16
NW = NC * NS
L = 16

DP = 32
K = 768
KB = K // 128
NCHUNK = 261
EW = NCHUNK * K
E_PAD = NS * EW
NACC = 100352
HALF = NACC // 2
NACCH = 50304
DUMMY = HALF
SLAB = NACCH // NS

B_GATHER = 24576
GPW = B_GATHER // NW
GB = GPW // 128


def _spmm_body(table, srcb, dstb, wflat, zeros, out,
               idx_v, dsti_v, w_v, rows_v, acc, sem_g, sem_s):
    cid = lax.axis_index("c")
    sid = lax.axis_index("s")
    lanes = lax.iota(jnp.int32, 16)
    lo = cid * HALF

    pltpu.sync_copy(zeros.at[pl.ds(sid * SLAB, SLAB)],
                    acc.at[pl.ds(sid * SLAB, SLAB)])
    plsc.subcore_barrier()

    def chunk_body(ck, carry):
        row0 = sid * (NCHUNK * KB) + ck * KB
        e0 = sid * EW + ck * K
        pltpu.sync_copy(srcb.at[pl.ds(row0, KB)], idx_v)
        pltpu.sync_copy(dstb.at[pl.ds(row0, KB)], dsti_v)
        pltpu.sync_copy(wflat.at[pl.ds(e0, K)], w_v)
        descs = [
            pltpu.async_copy(table.at[idx_v.at[j]],
                             rows_v.at[pl.ds(j * 128, 128)], sem_g)
            for j in range(KB)
        ]
        for dsc in descs:
            dsc.wait()

        def scale_body(g, c2):
            eb = g * 16
            r = g // 8
            cb = (g % 8) * 16
            dvec = dsti_v[r, pl.ds(cb, 16)]
            loc = dvec - lo
            ok = (loc >= 0) & (loc < HALF)
            spread = DUMMY + (dvec & 127)
            dsti_v[r, pl.ds(cb, 16)] = jnp.where(ok, loc, spread)
            wv16 = w_v[pl.ds(eb, 16)]
            dnums = lax.GatherDimensionNumbers(
                offset_dims=(), collapsed_slice_dims=(0,),
                start_index_map=(0,))
            for i in range(16):
                ws = lax.gather(wv16, jnp.full((16, 1), i, jnp.int32),
                                dnums, (1,),
                                mode=lax.GatherScatterMode.PROMISE_IN_BOUNDS)
                e = eb + i
                v0 = rows_v[e, pl.ds(0, 16)]
                v1 = rows_v[e, pl.ds(16, 16)]
                rows_v[e, pl.ds(0, 16)] = v0 * ws
                rows_v[e, pl.ds(16, 16)] = v1 * ws
            return c2

        lax.fori_loop(0, K // 16, scale_body, 0)

        descs2 = [
            pltpu.async_copy(rows_v.at[pl.ds(j * 128, 128)],
                             acc.at[dsti_v.at[j]], sem_s, add=True)
            for j in range(KB)
        ]
        for dsc in descs2:
            dsc.wait()
        return carry

    lax.fori_loop(0, NCHUNK, chunk_body, 0)
    plsc.subcore_barrier()
    pltpu.sync_copy(acc.at[pl.ds(sid * SLAB, SLAB)],
                    out.at[cid, pl.ds(sid * SLAB, SLAB)])


_spmm = pl.kernel(
    _spmm_body,
    out_type=jax.ShapeDtypeStruct((NC, NACCH, DP), jnp.float32),
    mesh=plsc.VectorSubcoreMesh(core_axis_name="c", subcore_axis_name="s"),
    compiler_params=pltpu.CompilerParams(needs_layout_passes=False,
                                         use_tc_tiling_on_sc=False),
    scratch_types=[
        pltpu.VMEM((KB, 128), jnp.int32),
        pltpu.VMEM((KB, 128), jnp.int32),
        pltpu.VMEM((K,), jnp.float32),
        pltpu.VMEM((K, DP), jnp.float32),
        pltpu.VMEM_SHARED((NACCH, DP), jnp.float32),
        pltpu.SemaphoreType.DMA,
        pltpu.SemaphoreType.DMA,
    ],
)


def _gather_body(light, idxb, out, idx_v, rows_v, sem):
    cid = lax.axis_index("c")
    sid = lax.axis_index("s")
    wid = cid * NS + sid
    pltpu.sync_copy(idxb.at[pl.ds(wid * GB, GB)], idx_v)
    descs = [
        pltpu.async_copy(light.at[idx_v.at[j]],
                         rows_v.at[pl.ds(j * 128, 128)], sem)
        for j in range(GB)
    ]
    for dsc in descs:
        dsc.wait()
    pltpu.sync_copy(rows_v, out.at[pl.ds(wid * GPW, GPW)])


_gather = pl.kernel(
    _gather_body,
    out_type=jax.ShapeDtypeStruct((B_GATHER, DP), jnp.float32),
    mesh=plsc.VectorSubcoreMesh(core_axis_name="c", subcore_axis_name="s"),
    compiler_params=pltpu.CompilerParams(needs_layout_passes=False, use_tc_tiling_on_sc=False),
    scratch_types=[
        pltpu.VMEM((GB, 128), jnp.int32),
        pltpu.VMEM((GPW, DP), jnp.float32),
        pltpu.SemaphoreType.DMA,
    ],
)


def _add2_body(p_ref, r_ref, t_ref, ro_ref):
    s = p_ref[0]
    t_ref[...] = s
    ro_ref[...] = r_ref[...] + s


def _addfin_body(p_ref, r_ref, o_ref):
    o_ref[...] = (r_ref[...] + p_ref[0]) * 0.25


_ADD_BR = 784
_ADD_GRID = NACC // _ADD_BR
_HB = HALF // _ADD_BR


def _add2(partials, run):
    return pl.pallas_call(
        _add2_body,
        grid=(_ADD_GRID,),
        in_specs=[
            pl.BlockSpec((1, _ADD_BR, DP), lambda i: (i // _HB, i % _HB, 0)),
            pl.BlockSpec((_ADD_BR, DP), lambda i: (i, 0)),
        ],
        out_specs=[
            pl.BlockSpec((_ADD_BR, DP), lambda i: (i, 0)),
            pl.BlockSpec((_ADD_BR, DP), lambda i: (i, 0)),
        ],
        out_shape=[
            jax.ShapeDtypeStruct((NACC, DP), jnp.float32),
            jax.ShapeDtypeStruct((NACC, DP), jnp.float32),
        ],
    )(partials, run)


def _addfin(partials, run):
    return pl.pallas_call(
        _addfin_body,
        grid=(_ADD_GRID,),
        in_specs=[
            pl.BlockSpec((1, _ADD_BR, DP), lambda i: (i // _HB, i % _HB, 0)),
            pl.BlockSpec((_ADD_BR, DP), lambda i: (i, 0)),
        ],
        out_specs=pl.BlockSpec((_ADD_BR, DP), lambda i: (i, 0)),
        out_shape=jax.ShapeDtypeStruct((NACC, DP), jnp.float32),
    )(partials, run)


def _mlp_body(x_ref, w1_ref, b1_ref, w2_ref, b2_ref, o_ref):
    x = x_ref[...]
    n = jnp.sqrt(jnp.sum(x * x, axis=1, keepdims=True))
    f = x / jnp.maximum(n, 1e-12)
    h = jnp.dot(f, w1_ref[...], preferred_element_type=jnp.float32)
    h = h + b1_ref[...]
    h = jnp.where(h >= 0, h, 0.2 * h)
    n2 = jnp.sqrt(jnp.sum(h * h, axis=1, keepdims=True))
    g = h / jnp.maximum(n2, 1e-12)
    o_ref[...] = jnp.dot(g, w2_ref[...],
                         preferred_element_type=jnp.float32) + b2_ref[...]


def _mlp(x, w1, b1, w2, b2, bm=512):
    rows = x.shape[0]
    return pl.pallas_call(
        _mlp_body,
        grid=(rows // bm,),
        in_specs=[
            pl.BlockSpec((bm, 1024), lambda i: (i, 0)),
            pl.BlockSpec((1024, HID), lambda i: (0, 0)),
            pl.BlockSpec((1, HID), lambda i: (0, 0)),
            pl.BlockSpec((HID, D), lambda i: (0, 0)),
            pl.BlockSpec((1, D), lambda i: (0, 0)),
        ],
        out_specs=pl.BlockSpec((bm, D), lambda i: (i, 0)),
        out_shape=jax.ShapeDtypeStruct((rows, D), jnp.float32),
    )(x, w1, b1.reshape(1, -1), w2, b2.reshape(1, -1))


def kernel(user_positive_items_pairs, negative_samples, textual_feature_pos,
           visual_feature_pos, textual_feature_neg, visual_feature_neg,
           user_table, item_table, edge_index, edge_weight,
           t1_W, t1_b, t2_W, t2_b, v1_W, v1_b, v2_W, v2_b):
    emb0 = jnp.concatenate(
        [user_table, item_table,
         jnp.zeros((NACC - N_NODES, D), jnp.float32)], axis=0)
    emb0 = jnp.pad(emb0, ((0, 0), (0, DP - D)))

    pad = E_PAD - N_EDGES
    srcb = jnp.concatenate(
        [edge_index[0], jnp.zeros((pad,), jnp.int32)]).reshape(-1, 128)
    dstb = jnp.concatenate(
        [edge_index[1], jnp.full((pad,), N_NODES, jnp.int32)]).reshape(-1, 128)
    wflat = jnp.concatenate([edge_weight, jnp.zeros((pad,), jnp.float32)])
    zeros = jnp.zeros((NACCH, DP), jnp.float32)

    table = emb0
    run = emb0
    light = None
    for layer in range(N_LAYERS):
        partials = _spmm(table, srcb, dstb, wflat, zeros)
        if layer < N_LAYERS - 1:
            table, run = _add2(partials, run)
        else:
            light = _addfin(partials, run)

    u_idx = user_positive_items_pairs[:, 0]
    p_idx = user_positive_items_pairs[:, 1] + N_USERS
    n_idx = negative_samples.reshape(-1) + N_USERS
    idx_all = jnp.concatenate([u_idx, p_idx, n_idx]).reshape(-1, 128)
    g24 = _gather(light, idx_all)

    g24 = g24[:, :D]
    users = g24[:4096]
    pos_items = g24[4096:8192]
    neg_items = g24[8192:]

    pos_t = _mlp(textual_feature_pos, t1_W, t1_b, t2_W, t2_b)
    neg_t = _mlp(textual_feature_neg.reshape(-1, 1024), t1_W, t1_b, t2_W, t2_b)
    pos_v = _mlp(visual_feature_pos, v1_W, v1_b, v2_W, v2_b)
    neg_v = _mlp(visual_feature_neg.reshape(-1, 1024), v1_W, v1_b, v2_W, v2_b)

    items = jnp.concatenate([pos_items, neg_items], axis=0)
    textual_f = jnp.concatenate([pos_t, neg_t], axis=0)
    visual_f = jnp.concatenate([pos_v, neg_v], axis=0)
    user_a_ = jnp.repeat(users[:, None, :], NUM_NEG, axis=1).reshape(-1, D)
    users_all = jnp.concatenate([users, user_a_], axis=0)
    return jnp.stack([users_all, items, textual_f, visual_f], axis=0)


# speedup vs baseline: 12.1039x; 1.0982x over previous
import functools

import jax
import jax.numpy as jnp
from jax import lax
from jax.experimental import pallas as pl
from jax.experimental.pallas import tpu as pltpu
from jax.experimental.pallas import tpu_sc as plsc

N_USERS = 50000
N_ITEMS = 50000
N_NODES = N_USERS + N_ITEMS
N_EDGES = 3200000
D = 20
NUM_NEG = 4
N_LAYERS = 3
HID = 512

NC = 2
NS = 16
NW = NC * NS
L = 16

DP = 32
K = 768
KB = K // 128
NCHUNK = 262
EW = NCHUNK * K
E_PAD = NS * EW
E_ALLOC = E_PAD + K
NACC = 100352
HALF = NACC // 2
NACCH = 50304
DUMMY = HALF
SLAB = NACCH // NS

B_GATHER = 24576
GPW = B_GATHER // NW
GB = GPW // 128


def _spmm_body(table, srcb, dstb, wflat, zeros, out,
               idxA, dstA, wA, idxB, dstB, wB, rows_v, acc,
               sem_p, sem_g, sem_s):
    cid = lax.axis_index("c")
    sid = lax.axis_index("s")
    lanes = lax.iota(jnp.int32, 16)
    lo = cid * HALF
    base_r = sid * (NCHUNK * KB)
    base_e = sid * EW

    pltpu.sync_copy(zeros.at[pl.ds(sid * SLAB, SLAB)],
                    acc.at[pl.ds(sid * SLAB, SLAB)])
    plsc.subcore_barrier()

    def prefetch(ck, idx_b, dst_b, w_b):
        row0 = base_r + ck * KB
        e0 = base_e + ck * K
        return [
            pltpu.async_copy(srcb.at[pl.ds(row0, KB)], idx_b, sem_p),
            pltpu.async_copy(dstb.at[pl.ds(row0, KB)], dst_b, sem_p),
            pltpu.async_copy(wflat.at[pl.ds(e0, K)], w_b, sem_p),
        ]

    def proc(idx_b, dst_b, w_b):
        descs = [
            pltpu.async_copy(table.at[idx_b.at[j]],
                             rows_v.at[pl.ds(j * 128, 128)], sem_g)
            for j in range(KB)
        ]
        for dsc in descs:
            dsc.wait()

        def scale_body(g, c2):
            eb = g * 16
            r = g // 8
            cb = (g % 8) * 16
            dvec = dst_b[r, pl.ds(cb, 16)]
            loc = dvec - lo
            ok = (loc >= 0) & (loc < HALF)
            spread = DUMMY + (dvec & 127)
            dst_b[r, pl.ds(cb, 16)] = jnp.where(ok, loc, spread)
            wv16 = w_b[pl.ds(eb, 16)]
            dnums = lax.GatherDimensionNumbers(
                offset_dims=(), collapsed_slice_dims=(0,),
                start_index_map=(0,))
            for i in range(16):
                ws = lax.gather(wv16, jnp.full((16, 1), i, jnp.int32),
                                dnums, (1,),
                                mode=lax.GatherScatterMode.PROMISE_IN_BOUNDS)
                e = eb + i
                v0 = rows_v[e, pl.ds(0, 16)]
                v1 = rows_v[e, pl.ds(16, 16)]
                rows_v[e, pl.ds(0, 16)] = v0 * ws
                rows_v[e, pl.ds(16, 16)] = v1 * ws
            return c2

        lax.fori_loop(0, K // 16, scale_body, 0)

        descs2 = [
            pltpu.async_copy(rows_v.at[pl.ds(j * 128, 128)],
                             acc.at[dst_b.at[j]], sem_s, add=True)
            for j in range(KB)
        ]
        for dsc in descs2:
            dsc.wait()

    for dsc in prefetch(0, idxA, dstA, wA):
        dsc.wait()

    def pair_body(i, carry):
        ckA = 2 * i
        dB = prefetch(ckA + 1, idxB, dstB, wB)
        proc(idxA, dstA, wA)
        for dsc in dB:
            dsc.wait()
        dA = prefetch(ckA + 2, idxA, dstA, wA)
        proc(idxB, dstB, wB)
        for dsc in dA:
            dsc.wait()
        return carry

    lax.fori_loop(0, NCHUNK // 2, pair_body, 0)
    plsc.subcore_barrier()
    pltpu.sync_copy(acc.at[pl.ds(sid * SLAB, SLAB)],
                    out.at[cid, pl.ds(sid * SLAB, SLAB)])


_spmm = pl.kernel(
    _spmm_body,
    out_type=jax.ShapeDtypeStruct((NC, NACCH, DP), jnp.float32),
    mesh=plsc.VectorSubcoreMesh(core_axis_name="c", subcore_axis_name="s"),
    compiler_params=pltpu.CompilerParams(needs_layout_passes=False,
                                         use_tc_tiling_on_sc=False),
    scratch_types=[
        pltpu.VMEM((KB, 128), jnp.int32),
        pltpu.VMEM((KB, 128), jnp.int32),
        pltpu.VMEM((K,), jnp.float32),
        pltpu.VMEM((KB, 128), jnp.int32),
        pltpu.VMEM((KB, 128), jnp.int32),
        pltpu.VMEM((K,), jnp.float32),
        pltpu.VMEM((K, DP), jnp.float32),
        pltpu.VMEM_SHARED((NACCH, DP), jnp.float32),
        pltpu.SemaphoreType.DMA,
        pltpu.SemaphoreType.DMA,
        pltpu.SemaphoreType.DMA,
    ],
)


def _gather_body(light, idxb, out, idx_v, rows_v, sem):
    cid = lax.axis_index("c")
    sid = lax.axis_index("s")
    wid = cid * NS + sid
    pltpu.sync_copy(idxb.at[pl.ds(wid * GB, GB)], idx_v)
    descs = [
        pltpu.async_copy(light.at[idx_v.at[j]],
                         rows_v.at[pl.ds(j * 128, 128)], sem)
        for j in range(GB)
    ]
    for dsc in descs:
        dsc.wait()
    pltpu.sync_copy(rows_v, out.at[pl.ds(wid * GPW, GPW)])


_gather = pl.kernel(
    _gather_body,
    out_type=jax.ShapeDtypeStruct((B_GATHER, DP), jnp.float32),
    mesh=plsc.VectorSubcoreMesh(core_axis_name="c", subcore_axis_name="s"),
    compiler_params=pltpu.CompilerParams(needs_layout_passes=False, use_tc_tiling_on_sc=False),
    scratch_types=[
        pltpu.VMEM((GB, 128), jnp.int32),
        pltpu.VMEM((GPW, DP), jnp.float32),
        pltpu.SemaphoreType.DMA,
    ],
)


def _add2_body(p_ref, r_ref, t_ref, ro_ref):
    s = p_ref[0]
    t_ref[...] = s
    ro_ref[...] = r_ref[...] + s


def _addfin_body(p_ref, r_ref, o_ref):
    o_ref[...] = (r_ref[...] + p_ref[0]) * 0.25


_ADD_BR = 784
_ADD_GRID = NACC // _ADD_BR
_HB = HALF // _ADD_BR


def _add2(partials, run):
    return pl.pallas_call(
        _add2_body,
        grid=(_ADD_GRID,),
        in_specs=[
            pl.BlockSpec((1, _ADD_BR, DP), lambda i: (i // _HB, i % _HB, 0)),
            pl.BlockSpec((_ADD_BR, DP), lambda i: (i, 0)),
        ],
        out_specs=[
            pl.BlockSpec((_ADD_BR, DP), lambda i: (i, 0)),
            pl.BlockSpec((_ADD_BR, DP), lambda i: (i, 0)),
        ],
        out_shape=[
            jax.ShapeDtypeStruct((NACC, DP), jnp.float32),
            jax.ShapeDtypeStruct((NACC, DP), jnp.float32),
        ],
    )(partials, run)


def _addfin(partials, run):
    return pl.pallas_call(
        _addfin_body,
        grid=(_ADD_GRID,),
        in_specs=[
            pl.BlockSpec((1, _ADD_BR, DP), lambda i: (i // _HB, i % _HB, 0)),
            pl.BlockSpec((_ADD_BR, DP), lambda i: (i, 0)),
        ],
        out_specs=pl.BlockSpec((_ADD_BR, DP), lambda i: (i, 0)),
        out_shape=jax.ShapeDtypeStruct((NACC, DP), jnp.float32),
    )(partials, run)


def _mlp_body(x_ref, w1_ref, b1_ref, w2_ref, b2_ref, o_ref):
    x = x_ref[...]
    n = jnp.sqrt(jnp.sum(x * x, axis=1, keepdims=True))
    f = x / jnp.maximum(n, 1e-12)
    h = jnp.dot(f, w1_ref[...], preferred_element_type=jnp.float32)
    h = h + b1_ref[...]
    h = jnp.where(h >= 0, h, 0.2 * h)
    n2 = jnp.sqrt(jnp.sum(h * h, axis=1, keepdims=True))
    g = h / jnp.maximum(n2, 1e-12)
    o_ref[...] = jnp.dot(g, w2_ref[...],
                         preferred_element_type=jnp.float32) + b2_ref[...]


def _mlp(x, w1, b1, w2, b2, bm=512):
    rows = x.shape[0]
    return pl.pallas_call(
        _mlp_body,
        grid=(rows // bm,),
        in_specs=[
            pl.BlockSpec((bm, 1024), lambda i: (i, 0)),
            pl.BlockSpec((1024, HID), lambda i: (0, 0)),
            pl.BlockSpec((1, HID), lambda i: (0, 0)),
            pl.BlockSpec((HID, D), lambda i: (0, 0)),
            pl.BlockSpec((1, D), lambda i: (0, 0)),
        ],
        out_specs=pl.BlockSpec((bm, D), lambda i: (i, 0)),
        out_shape=jax.ShapeDtypeStruct((rows, D), jnp.float32),
    )(x, w1, b1.reshape(1, -1), w2, b2.reshape(1, -1))


def kernel(user_positive_items_pairs, negative_samples, textual_feature_pos,
           visual_feature_pos, textual_feature_neg, visual_feature_neg,
           user_table, item_table, edge_index, edge_weight,
           t1_W, t1_b, t2_W, t2_b, v1_W, v1_b, v2_W, v2_b):
    emb0 = jnp.concatenate(
        [user_table, item_table,
         jnp.zeros((NACC - N_NODES, D), jnp.float32)], axis=0)
    emb0 = jnp.pad(emb0, ((0, 0), (0, DP - D)))

    pad = E_ALLOC - N_EDGES
    srcb = jnp.concatenate(
        [edge_index[0], jnp.zeros((pad,), jnp.int32)]).reshape(-1, 128)
    dstb = jnp.concatenate(
        [edge_index[1], jnp.full((pad,), N_NODES, jnp.int32)]).reshape(-1, 128)
    wflat = jnp.concatenate([edge_weight, jnp.zeros((pad,), jnp.float32)])
    zeros = jnp.zeros((NACCH, DP), jnp.float32)

    table = emb0
    run = emb0
    light = None
    for layer in range(N_LAYERS):
        partials = _spmm(table, srcb, dstb, wflat, zeros)
        if layer < N_LAYERS - 1:
            table, run = _add2(partials, run)
        else:
            light = _addfin(partials, run)

    u_idx = user_positive_items_pairs[:, 0]
    p_idx = user_positive_items_pairs[:, 1] + N_USERS
    n_idx = negative_samples.reshape(-1) + N_USERS
    idx_all = jnp.concatenate([u_idx, p_idx, n_idx]).reshape(-1, 128)
    g24 = _gather(light, idx_all)

    g24 = g24[:, :D]
    users = g24[:4096]
    pos_items = g24[4096:8192]
    neg_items = g24[8192:]

    pos_t = _mlp(textual_feature_pos, t1_W, t1_b, t2_W, t2_b)
    neg_t = _mlp(textual_feature_neg.reshape(-1, 1024), t1_W, t1_b, t2_W, t2_b)
    pos_v = _mlp(visual_feature_pos, v1_W, v1_b, v2_W, v2_b)
    neg_v = _mlp(visual_feature_neg.reshape(-1, 1024), v1_W, v1_b, v2_W, v2_b)

    items = jnp.concatenate([pos_items, neg_items], axis=0)
    textual_f = jnp.concatenate([pos_t, neg_t], axis=0)
    visual_f = jnp.concatenate([pos_v, neg_v], axis=0)
    user_a_ = jnp.repeat(users[:, None, :], NUM_NEG, axis=1).reshape(-1, D)
    users_all = jnp.concatenate([users, user_a_], axis=0)
    return jnp.stack([users_all, items, textual_f, visual_f], axis=0)
